# baseline (speedup 1.0000x reference)
import functools

import numpy as np
import jax
import jax.numpy as jnp
from jax import lax
from jax.experimental import pallas as pl
from jax.experimental.pallas import tpu as pltpu
from jax.experimental.pallas import tpu_sc as plsc

VOCAB = 100000
EMBED = 64
HIDDEN = 128
OUT = 2
B = 4096
L = 200

NC = 2
NS = 16
NW = NC * NS
BPW = B // NW
CHUNK = 2
CL = CHUNK * L
NCHUNKS = BPW // CHUNK
UNROLL = 8


def _pooled_sum_sc(text, emb_bf):
    mesh = plsc.VectorSubcoreMesh(core_axis_name="c", subcore_axis_name="s")

    @functools.partial(
        pl.kernel,
        mesh=mesh,
        out_type=jax.ShapeDtypeStruct((B, EMBED), jnp.float32),
        scratch_types=[
            pltpu.VMEM((BPW, L), jnp.int32),
            pltpu.VMEM((CL, EMBED // 2), jnp.int32),
            pltpu.VMEM((CL, EMBED // 2), jnp.int32),
            pltpu.VMEM((BPW, EMBED), jnp.float32),
            pltpu.SemaphoreType.DMA,
            pltpu.SemaphoreType.DMA,
        ],
        compiler_params=pltpu.CompilerParams(
            use_tc_tiling_on_sc=False, needs_layout_passes=False),
    )
    def body(text_hbm, emb_hbm, out_hbm, idx_v, rows_a, rows_b, out_v,
             sem_a, sem_b):
        wid = lax.axis_index("s") * NC + lax.axis_index("c")
        base_row = wid * BPW
        rows = (rows_a, rows_b)
        sems = (sem_a, sem_b)

        pltpu.sync_copy(text_hbm.at[pl.ds(base_row, BPW), :], idx_v)

        def gather(par, ci):
            return [
                pltpu.make_async_copy(
                    emb_hbm.at[idx_v.at[ci * CHUNK + c]],
                    rows[par].at[pl.ds(c * L, L)],
                    sems[par])
                for c in range(CHUNK)
            ]

        for d in gather(0, 0):
            d.start()

        def step_body(half, carry):
            for par in range(2):
                ci = half * 2 + par

                @pl.when(ci + 1 < NCHUNKS)
                def _():
                    for d in gather(1 - par, ci + 1):
                        d.start()

                for d in gather(par, ci):
                    d.wait()
                rows_v = rows[par]
                for c in range(CHUNK):
                    def sum_body(j, acc):
                        a0, a1, a2, a3 = acc
                        r0 = c * L + j * UNROLL
                        for u in range(UNROLL):
                            r = r0 + u
                            lo = plsc.bitcast(
                                rows_v[r, pl.ds(0, 16)], jnp.bfloat16)
                            hi = plsc.bitcast(
                                rows_v[r, pl.ds(16, 16)], jnp.bfloat16)
                            e0, o0 = plsc.unpack(
                                lo, format=plsc.PackFormat.INTERLEAVED)
                            e1, o1 = plsc.unpack(
                                hi, format=plsc.PackFormat.INTERLEAVED)
                            a0 = a0 + e0
                            a1 = a1 + o0
                            a2 = a2 + e1
                            a3 = a3 + o1
                        return (a0, a1, a2, a3)

                    zero = jnp.zeros((16,), jnp.float32)
                    a0, a1, a2, a3 = lax.fori_loop(
                        0, L // UNROLL, sum_body, (zero, zero, zero, zero))
                    out_v[ci * CHUNK + c, pl.ds(0, 16)] = a0
                    out_v[ci * CHUNK + c, pl.ds(16, 16)] = a1
                    out_v[ci * CHUNK + c, pl.ds(32, 16)] = a2
                    out_v[ci * CHUNK + c, pl.ds(48, 16)] = a3
            return carry

        lax.fori_loop(0, NCHUNKS // 2, step_body, 0)

        pltpu.sync_copy(out_v, out_hbm.at[pl.ds(base_row, BPW)])

    return body(text, emb_bf)


_PERM = np.concatenate([
    np.arange(0, 32, 2),
    np.arange(1, 32, 2),
    np.arange(32, 64, 2),
    np.arange(33, 64, 2),
])


def _cast_body(x_ref, o_ref):
    o_ref[...] = x_ref[...].astype(jnp.bfloat16)


def _cast_tc(emb):
    BV = VOCAB // 10
    return pl.pallas_call(
        _cast_body,
        grid=(10,),
        in_specs=[pl.BlockSpec((BV, EMBED), lambda i: (i, 0))],
        out_specs=pl.BlockSpec((BV, EMBED), lambda i: (i, 0)),
        out_shape=jax.ShapeDtypeStruct((VOCAB, EMBED), jnp.bfloat16),
    )(emb)


def _mlp_body(sum_ref, len_ref, w1_ref, b1_ref, w2_ref, b2_ref, out_ref):
    lens = jnp.maximum(len_ref[...].astype(jnp.float32), 1.0)
    x = sum_ref[...] * (1.0 / lens)
    h = jnp.dot(x, w1_ref[...], preferred_element_type=jnp.float32)
    h = jnp.maximum(h + b1_ref[...], 0.0)
    o = jnp.dot(h, w2_ref[...], preferred_element_type=jnp.float32)
    out_ref[...] = o + b2_ref[...]


def _mlp_tc(summed, lengths, W1t, b1, W2t, b2):
    BT = 512
    grid = (B // BT,)
    return pl.pallas_call(
        _mlp_body,
        grid=grid,
        in_specs=[
            pl.BlockSpec((BT, EMBED), lambda i: (i, 0)),
            pl.BlockSpec((BT, 1), lambda i: (i, 0)),
            pl.BlockSpec((EMBED, HIDDEN), lambda i: (0, 0)),
            pl.BlockSpec((1, HIDDEN), lambda i: (0, 0)),
            pl.BlockSpec((HIDDEN, OUT), lambda i: (0, 0)),
            pl.BlockSpec((1, OUT), lambda i: (0, 0)),
        ],
        out_specs=pl.BlockSpec((BT, OUT), lambda i: (i, 0)),
        out_shape=jax.ShapeDtypeStruct((B, OUT), jnp.float32),
    )(summed, lengths, W1t, b1, W2t, b2)


def kernel(text, lengths, emb, W1, b1, W2, b2):
    emb_packed = jax.lax.bitcast_convert_type(
        emb.astype(jnp.bfloat16).reshape(VOCAB, EMBED // 2, 2), jnp.int32)
    summed = _pooled_sum_sc(text, emb_packed)
    return _mlp_tc(
        summed,
        lengths.reshape(B, 1),
        W1.T[_PERM, :],
        b1.reshape(1, HIDDEN),
        W2.T,
        b2.reshape(1, OUT),
    )


# speedup vs baseline: 1.6738x; 1.6738x over previous
import functools

import numpy as np
import jax
import jax.numpy as jnp
from jax import lax
from jax.experimental import pallas as pl
from jax.experimental.pallas import tpu as pltpu
from jax.experimental.pallas import tpu_sc as plsc

VOCAB = 100000
EMBED = 64
HIDDEN = 128
OUT = 2
B = 4096
L = 200

NC = 2
NS = 16
NW = NC * NS
BPW = B // NW
CHUNK = 2
CL = CHUNK * L
NCHUNKS = BPW // CHUNK
UNROLL = 8


def _pooled_sum_sc(text, emb_bf):
    mesh = plsc.VectorSubcoreMesh(core_axis_name="c", subcore_axis_name="s")

    @functools.partial(
        pl.kernel,
        mesh=mesh,
        out_type=jax.ShapeDtypeStruct((B, EMBED), jnp.float32),
        scratch_types=[
            pltpu.VMEM((BPW, L), jnp.int32),
            pltpu.VMEM((CL, EMBED // 2), jnp.float32),
            pltpu.VMEM((CL, EMBED // 2), jnp.float32),
            pltpu.VMEM((BPW, EMBED), jnp.float32),
            pltpu.SemaphoreType.DMA,
            pltpu.SemaphoreType.DMA,
        ],
        compiler_params=pltpu.CompilerParams(
            use_tc_tiling_on_sc=False, needs_layout_passes=False),
    )
    def body(text_hbm, emb_hbm, out_hbm, idx_v, rows_a, rows_b, out_v,
             sem_a, sem_b):
        wid = lax.axis_index("s") * NC + lax.axis_index("c")
        base_row = wid * BPW
        rows = (rows_a, rows_b)
        sems = (sem_a, sem_b)

        pltpu.sync_copy(text_hbm.at[pl.ds(base_row, BPW), :], idx_v)

        def gather(par, ci):
            return [
                pltpu.make_async_copy(
                    emb_hbm.at[idx_v.at[ci * CHUNK + c]],
                    rows[par].at[pl.ds(c * L, L)],
                    sems[par])
                for c in range(CHUNK)
            ]

        for d in gather(0, 0):
            d.start()

        def step_body(half, carry):
            for par in range(2):
                ci = half * 2 + par

                @pl.when(ci + 1 < NCHUNKS)
                def _():
                    for d in gather(1 - par, ci + 1):
                        d.start()

                for d in gather(par, ci):
                    d.wait()
                rows_v = rows[par]
                for c in range(CHUNK):
                    def sum_body(j, acc):
                        a0, a1, a2, a3 = acc
                        r0 = c * L + j * UNROLL
                        for u in range(UNROLL):
                            r = r0 + u
                            lo = plsc.bitcast(
                                rows_v[r, pl.ds(0, 16)], jnp.bfloat16)
                            hi = plsc.bitcast(
                                rows_v[r, pl.ds(16, 16)], jnp.bfloat16)
                            e0, o0 = plsc.unpack(
                                lo, format=plsc.PackFormat.INTERLEAVED)
                            e1, o1 = plsc.unpack(
                                hi, format=plsc.PackFormat.INTERLEAVED)
                            a0 = a0 + e0
                            a1 = a1 + o0
                            a2 = a2 + e1
                            a3 = a3 + o1
                        return (a0, a1, a2, a3)

                    zero = jnp.zeros((16,), jnp.float32)
                    a0, a1, a2, a3 = lax.fori_loop(
                        0, L // UNROLL, sum_body, (zero, zero, zero, zero))
                    out_v[ci * CHUNK + c, pl.ds(0, 16)] = a0
                    out_v[ci * CHUNK + c, pl.ds(16, 16)] = a1
                    out_v[ci * CHUNK + c, pl.ds(32, 16)] = a2
                    out_v[ci * CHUNK + c, pl.ds(48, 16)] = a3
            return carry

        lax.fori_loop(0, NCHUNKS // 2, step_body, 0)

        pltpu.sync_copy(out_v, out_hbm.at[pl.ds(base_row, BPW)])

    return body(text, emb_bf)


_PERM = np.concatenate([
    np.arange(0, 32, 2),
    np.arange(1, 32, 2),
    np.arange(32, 64, 2),
    np.arange(33, 64, 2),
])


def _pack_body(x_ref, o_ref):
    x = x_ref[...]
    row = lax.broadcasted_iota(jnp.int32, (EMBED, EMBED // 2), 0)
    col = lax.broadcasted_iota(jnp.int32, (EMBED, EMBED // 2), 1)
    pe = (row == 2 * col).astype(jnp.float32)
    po = (row == 2 * col + 1).astype(jnp.float32)
    even = jnp.dot(x, pe, preferred_element_type=jnp.float32)
    odd = jnp.dot(x, po, preferred_element_type=jnp.float32)
    lo = lax.bitcast_convert_type(even.astype(jnp.bfloat16), jnp.uint16)
    hi = lax.bitcast_convert_type(odd.astype(jnp.bfloat16), jnp.uint16)
    word = (hi.astype(jnp.uint32) << 16) | lo.astype(jnp.uint32)
    o_ref[...] = lax.bitcast_convert_type(word, jnp.float32)


def _pack_tc(emb):
    BV = VOCAB // 10
    return pl.pallas_call(
        _pack_body,
        grid=(10,),
        in_specs=[pl.BlockSpec((BV, EMBED), lambda i: (i, 0))],
        out_specs=pl.BlockSpec((BV, EMBED // 2), lambda i: (i, 0)),
        out_shape=jax.ShapeDtypeStruct((VOCAB, EMBED // 2), jnp.float32),
    )(emb)


def _mlp_body(sum_ref, len_ref, w1_ref, b1_ref, w2_ref, b2_ref, out_ref):
    lens = jnp.maximum(len_ref[...].astype(jnp.float32), 1.0)
    x = sum_ref[...] * (1.0 / lens)
    h = jnp.dot(x, w1_ref[...], preferred_element_type=jnp.float32)
    h = jnp.maximum(h + b1_ref[...], 0.0)
    o = jnp.dot(h, w2_ref[...], preferred_element_type=jnp.float32)
    out_ref[...] = o + b2_ref[...]


def _mlp_tc(summed, lengths, W1t, b1, W2t, b2):
    BT = 512
    grid = (B // BT,)
    return pl.pallas_call(
        _mlp_body,
        grid=grid,
        in_specs=[
            pl.BlockSpec((BT, EMBED), lambda i: (i, 0)),
            pl.BlockSpec((BT, 1), lambda i: (i, 0)),
            pl.BlockSpec((EMBED, HIDDEN), lambda i: (0, 0)),
            pl.BlockSpec((1, HIDDEN), lambda i: (0, 0)),
            pl.BlockSpec((HIDDEN, OUT), lambda i: (0, 0)),
            pl.BlockSpec((1, OUT), lambda i: (0, 0)),
        ],
        out_specs=pl.BlockSpec((BT, OUT), lambda i: (i, 0)),
        out_shape=jax.ShapeDtypeStruct((B, OUT), jnp.float32),
    )(summed, lengths, W1t, b1, W2t, b2)


def kernel(text, lengths, emb, W1, b1, W2, b2):
    summed = _pooled_sum_sc(text, _pack_tc(emb))
    return _mlp_tc(
        summed,
        lengths.reshape(B, 1),
        W1.T[_PERM, :],
        b1.reshape(1, HIDDEN),
        W2.T,
        b2.reshape(1, OUT),
    )


# speedup vs baseline: 1.9208x; 1.1476x over previous
import functools

import jax
import jax.numpy as jnp
from jax import lax
from jax.experimental import pallas as pl
from jax.experimental.pallas import tpu as pltpu
from jax.experimental.pallas import tpu_sc as plsc

VOCAB = 100000
EMBED = 64
HIDDEN = 128
OUT = 2
B = 4096
L = 200

NC = 2
NS = 16
NW = NC * NS
BPW = B // NW
CHUNK = 2
CL = CHUNK * L
NCHUNKS = BPW // CHUNK
UNROLL = 8


def _pooled_sum_sc(text, emb):
    mesh = plsc.VectorSubcoreMesh(core_axis_name="c", subcore_axis_name="s")

    @functools.partial(
        pl.kernel,
        mesh=mesh,
        out_type=jax.ShapeDtypeStruct((B, EMBED), jnp.float32),
        scratch_types=[
            pltpu.VMEM((BPW, L), jnp.int32),
            pltpu.VMEM((CL, EMBED), jnp.float32),
            pltpu.VMEM((CL, EMBED), jnp.float32),
            pltpu.VMEM((BPW, EMBED), jnp.float32),
            pltpu.SemaphoreType.DMA,
            pltpu.SemaphoreType.DMA,
        ],
        compiler_params=pltpu.CompilerParams(use_tc_tiling_on_sc=False),
    )
    def body(text_hbm, emb_hbm, out_hbm, idx_v, rows_a, rows_b, out_v,
             sem_a, sem_b):
        wid = lax.axis_index("s") * NC + lax.axis_index("c")
        base_row = wid * BPW
        rows = (rows_a, rows_b)
        sems = (sem_a, sem_b)

        pltpu.sync_copy(text_hbm.at[pl.ds(base_row, BPW), :], idx_v)

        def gather(par, ci):
            return [
                pltpu.make_async_copy(
                    emb_hbm.at[idx_v.at[ci * CHUNK + c]],
                    rows[par].at[pl.ds(c * L, L)],
                    sems[par])
                for c in range(CHUNK)
            ]

        for d in gather(0, 0):
            d.start()

        def step_body(half, carry):
            for par in range(2):
                ci = half * 2 + par

                @pl.when(ci + 1 < NCHUNKS)
                def _():
                    for d in gather(1 - par, ci + 1):
                        d.start()

                for d in gather(par, ci):
                    d.wait()
                rows_v = rows[par]
                for c in range(CHUNK):
                    def sum_body(j, acc):
                        a0, a1, a2, a3 = acc
                        r0 = c * L + j * UNROLL
                        for u in range(UNROLL):
                            r = r0 + u
                            a0 = a0 + rows_v[r, pl.ds(0, 16)]
                            a1 = a1 + rows_v[r, pl.ds(16, 16)]
                            a2 = a2 + rows_v[r, pl.ds(32, 16)]
                            a3 = a3 + rows_v[r, pl.ds(48, 16)]
                        return (a0, a1, a2, a3)

                    zero = jnp.zeros((16,), jnp.float32)
                    a0, a1, a2, a3 = lax.fori_loop(
                        0, L // UNROLL, sum_body, (zero, zero, zero, zero))
                    out_v[ci * CHUNK + c, pl.ds(0, 16)] = a0
                    out_v[ci * CHUNK + c, pl.ds(16, 16)] = a1
                    out_v[ci * CHUNK + c, pl.ds(32, 16)] = a2
                    out_v[ci * CHUNK + c, pl.ds(48, 16)] = a3
            return carry

        lax.fori_loop(0, NCHUNKS // 2, step_body, 0)

        pltpu.sync_copy(out_v, out_hbm.at[pl.ds(base_row, BPW)])

    return body(text, emb)


def _mlp_body(sum_ref, len_ref, w1_ref, b1_ref, w2_ref, b2_ref, out_ref):
    lens = jnp.maximum(len_ref[...].astype(jnp.float32), 1.0)
    x = sum_ref[...] * (1.0 / lens)
    h = jnp.dot(x, w1_ref[...], preferred_element_type=jnp.float32)
    h = jnp.maximum(h + b1_ref[...], 0.0)
    o = jnp.dot(h, w2_ref[...], preferred_element_type=jnp.float32)
    out_ref[...] = o + b2_ref[...]


def _mlp_tc(summed, lengths, W1t, b1, W2t, b2):
    BT = 2048
    grid = (B // BT,)
    return pl.pallas_call(
        _mlp_body,
        grid=grid,
        in_specs=[
            pl.BlockSpec((BT, EMBED), lambda i: (i, 0)),
            pl.BlockSpec((BT, 1), lambda i: (i, 0)),
            pl.BlockSpec((EMBED, HIDDEN), lambda i: (0, 0)),
            pl.BlockSpec((1, HIDDEN), lambda i: (0, 0)),
            pl.BlockSpec((HIDDEN, OUT), lambda i: (0, 0)),
            pl.BlockSpec((1, OUT), lambda i: (0, 0)),
        ],
        out_specs=pl.BlockSpec((BT, OUT), lambda i: (i, 0)),
        out_shape=jax.ShapeDtypeStruct((B, OUT), jnp.float32),
    )(summed, lengths, W1t, b1, W2t, b2)


def kernel(text, lengths, emb, W1, b1, W2, b2):
    summed = _pooled_sum_sc(text, emb)
    return _mlp_tc(
        summed,
        lengths.reshape(B, 1),
        W1.T,
        b1.reshape(1, HIDDEN),
        W2.T,
        b2.reshape(1, OUT),
    )


# speedup vs baseline: 1.9233x; 1.0013x over previous
import functools

import jax
import jax.numpy as jnp
from jax import lax
from jax.experimental import pallas as pl
from jax.experimental.pallas import tpu as pltpu
from jax.experimental.pallas import tpu_sc as plsc

VOCAB = 100000
EMBED = 64
HIDDEN = 128
OUT = 2
B = 4096
L = 200

NC = 2
NS = 16
NW = NC * NS
BPW = B // NW
CHUNK = 2
CL = CHUNK * L
NCHUNKS = BPW // CHUNK
UNROLL = 8


def _pooled_sum_sc(text, emb):
    mesh = plsc.VectorSubcoreMesh(core_axis_name="c", subcore_axis_name="s")

    @functools.partial(
        pl.kernel,
        mesh=mesh,
        out_type=jax.ShapeDtypeStruct((B, EMBED), jnp.float32),
        name="pooled_sum",
        scratch_types=[
            pltpu.VMEM((BPW, L), jnp.int32),
            pltpu.VMEM((CL, EMBED), jnp.float32),
            pltpu.VMEM((CL, EMBED), jnp.float32),
            pltpu.VMEM((BPW, EMBED), jnp.float32),
            pltpu.SemaphoreType.DMA,
            pltpu.SemaphoreType.DMA,
        ],
        compiler_params=pltpu.CompilerParams(use_tc_tiling_on_sc=False),
    )
    def body(text_hbm, emb_hbm, out_hbm, idx_v, rows_a, rows_b, out_v,
             sem_a, sem_b):
        wid = lax.axis_index("s") * NC + lax.axis_index("c")
        base_row = wid * BPW
        rows = (rows_a, rows_b)
        sems = (sem_a, sem_b)

        pltpu.sync_copy(text_hbm.at[pl.ds(base_row, BPW), :], idx_v)

        def gather(par, ci):
            return [
                pltpu.make_async_copy(
                    emb_hbm.at[idx_v.at[ci * CHUNK + c]],
                    rows[par].at[pl.ds(c * L, L)],
                    sems[par])
                for c in range(CHUNK)
            ]

        for d in gather(0, 0):
            d.start()

        def step_body(half, carry):
            for par in range(2):
                ci = half * 2 + par

                @pl.when(ci + 1 < NCHUNKS)
                def _():
                    for d in gather(1 - par, ci + 1):
                        d.start()

                for d in gather(par, ci):
                    d.wait()
                rows_v = rows[par]
                for c in range(CHUNK):
                    def sum_body(j, acc):
                        a0, a1, a2, a3 = acc
                        r0 = c * L + j * UNROLL
                        for u in range(UNROLL):
                            r = r0 + u
                            a0 = a0 + rows_v[r, pl.ds(0, 16)]
                            a1 = a1 + rows_v[r, pl.ds(16, 16)]
                            a2 = a2 + rows_v[r, pl.ds(32, 16)]
                            a3 = a3 + rows_v[r, pl.ds(48, 16)]
                        return (a0, a1, a2, a3)

                    zero = jnp.zeros((16,), jnp.float32)
                    a0, a1, a2, a3 = lax.fori_loop(
                        0, L // UNROLL, sum_body, (zero, zero, zero, zero))
                    out_v[ci * CHUNK + c, pl.ds(0, 16)] = a0
                    out_v[ci * CHUNK + c, pl.ds(16, 16)] = a1
                    out_v[ci * CHUNK + c, pl.ds(32, 16)] = a2
                    out_v[ci * CHUNK + c, pl.ds(48, 16)] = a3
            return carry

        lax.fori_loop(0, NCHUNKS // 2, step_body, 0)

        pltpu.sync_copy(out_v, out_hbm.at[pl.ds(base_row, BPW)])

    return body(text, emb)


def _mlp_body(sum_ref, len_ref, w1_ref, b1_ref, w2_ref, b2_ref, out_ref):
    lens = jnp.maximum(len_ref[...].astype(jnp.float32), 1.0)
    x = sum_ref[...] * (1.0 / lens)
    h = jnp.dot(x, w1_ref[...], preferred_element_type=jnp.float32)
    h = jnp.maximum(h + b1_ref[...], 0.0)
    o = jnp.dot(h, w2_ref[...], preferred_element_type=jnp.float32)
    out_ref[...] = o + b2_ref[...]


def _mlp_tc(summed, lengths, W1t, b1, W2t, b2):
    BT = 2048
    grid = (B // BT,)
    return pl.pallas_call(
        _mlp_body,
        grid=grid,
        in_specs=[
            pl.BlockSpec((BT, EMBED), lambda i: (i, 0)),
            pl.BlockSpec((BT, 1), lambda i: (i, 0)),
            pl.BlockSpec((EMBED, HIDDEN), lambda i: (0, 0)),
            pl.BlockSpec((1, HIDDEN), lambda i: (0, 0)),
            pl.BlockSpec((HIDDEN, OUT), lambda i: (0, 0)),
            pl.BlockSpec((1, OUT), lambda i: (0, 0)),
        ],
        out_specs=pl.BlockSpec((BT, OUT), lambda i: (i, 0)),
        out_shape=jax.ShapeDtypeStruct((B, OUT), jnp.float32),
    )(summed, lengths, W1t, b1, W2t, b2)


def kernel(text, lengths, emb, W1, b1, W2, b2):
    summed = _pooled_sum_sc(text, emb)
    return _mlp_tc(
        summed,
        lengths.reshape(B, 1),
        W1.T,
        b1.reshape(1, HIDDEN),
        W2.T,
        b2.reshape(1, OUT),
    )
